# baseline (speedup 1.0000x reference)
import functools

import jax
import jax.numpy as jnp
from jax import lax
from jax.experimental import pallas as pl
from jax.experimental.pallas import tpu as pltpu
from jax.experimental.pallas import tpu_sc as plsc

N = 10000
E = 320000
D = 128
C = 40
DP = 48
NP = 10240
NC = 2
NS = 16
NW = NC * NS
L = 16
B = 80
NCH = 125
RPS = NP // NS
RPW = NP // NW
SUB = 160
NBUF = 8
K = NBUF // 2
DEG_LAG = 8
DEGW = 16

_MESH = plsc.VectorSubcoreMesh(core_axis_name="c", subcore_axis_name="s")
_SC_PARAMS = pltpu.CompilerParams(use_tc_tiling_on_sc=False,
                                  needs_layout_passes=False)


def _fast_rsqrt(x):
    i = plsc.bitcast(x, jnp.int32)
    i = jnp.full((L,), 0x5F3759DF, jnp.int32) - lax.shift_right_logical(
        i, jnp.full((L,), 1, jnp.int32))
    r = plsc.bitcast(i, jnp.float32)
    r = r * (1.5 - 0.5 * x * r * r)
    r = r * (1.5 - 0.5 * x * r * r)
    r = r * (1.5 - 0.5 * x * r * r)
    return r


def _edge_pipeline(acc, src_all, dst_all, rows, gsem, ssem, table):
    def gissue(b, j):
        pltpu.async_copy(table.at[src_all.at[j]], rows[b], gsem[b])

    def gwait(b):
        pltpu.make_async_copy(table.at[src_all.at[0]], rows[b],
                              gsem[b]).wait()

    def sissue(b, j):
        pltpu.async_copy(rows[b], acc.at[dst_all.at[j]], ssem[b], add=True)

    def swait(b):
        pltpu.make_async_copy(rows[b], acc.at[dst_all.at[0]], ssem[b]).wait()

    for j in range(K):
        gissue(j % NBUF, j)
    for j in range(K):
        gwait(j % NBUF)
        sissue(j % NBUF, j)
        gissue((j + K) % NBUF, j + K)

    n_grp = (NCH - 2 * K) // NBUF

    def grp(g, carry):
        for bi in range(NBUF):
            j = K + g * NBUF + bi
            b = (K + bi) % NBUF
            gwait(b)
            sissue(b, j)
            swait(bi)
            gissue(bi, j + K)
        return carry

    lax.fori_loop(0, n_grp, grp, 0)

    for j in range(K + n_grp * NBUF, NCH - K):
        gwait(j % NBUF)
        sissue(j % NBUF, j)
        swait((j - K) % NBUF)
        gissue((j + K) % NBUF, j + K)
    for j in range(NCH - K, NCH):
        gwait(j % NBUF)
        sissue(j % NBUF, j)
        swait((j - K) % NBUF)
    for j in range(NCH - K, NCH):
        swait(j % NBUF)


def _hop1_body(dp_hbm, y_hbm, src_hbm, dst_hbm, z_hbm, parts_hbm, gout_hbm,
               acc, src_all, dst_all, a_v, y_v, dp0, dp1, *ring):
    _hop_common(True, dp_hbm, y_hbm, None, src_hbm, dst_hbm, z_hbm,
                parts_hbm, gout_hbm, acc, src_all, dst_all,
                a_v, y_v, None, dp0, dp1, ring)


def _hop2_body(dp_hbm, y_hbm, pp_hbm, src_hbm, dst_hbm, z_hbm, parts_hbm,
               gout_hbm, acc, src_all, dst_all, a_v, y_v, b_v, dp0, dp1,
               *ring):
    _hop_common(False, dp_hbm, y_hbm, pp_hbm, src_hbm, dst_hbm, z_hbm,
                parts_hbm, gout_hbm, acc, src_all, dst_all,
                a_v, y_v, b_v, dp0, dp1, ring)


def _hop_common(first, dp_hbm, y_hbm, pp_hbm, src_hbm, dst_hbm, z_hbm,
                parts_hbm, gout_hbm, acc, src_all, dst_all,
                a_v, y_v, b_v, dp0, dp1, ring):
    cid = lax.axis_index("c")
    sid = lax.axis_index("s")
    w = cid * NS + sid
    rows = ring[:NBUF]
    gsem = ring[NBUF:2 * NBUF]
    ssem = ring[2 * NBUF:]

    pltpu.sync_copy(z_hbm.at[pl.ds(sid * RPS, RPS)],
                    acc.at[pl.ds(sid * RPS, RPS)])
    pltpu.sync_copy(src_hbm.at[w], src_all)
    pltpu.sync_copy(dst_hbm.at[w], dst_all)

    for s in range(RPS // SUB):
        base = sid * RPS + s * SUB
        pltpu.sync_copy(dp_hbm.at[0, pl.ds(base, SUB)], dp0)
        pltpu.sync_copy(dp_hbm.at[1, pl.ds(base, SUB)], dp1)
        pltpu.sync_copy(y_hbm.at[pl.ds(base, SUB)], y_v)
        if not first:
            pltpu.sync_copy(pp_hbm.at[0, pl.ds(base, SUB)], a_v)
            pltpu.sync_copy(pp_hbm.at[1, pl.ds(base, SUB)], b_v)

        def row(r, carry):
            cnt = dp0[r, :] + dp1[r, :] + 1.0
            d = _fast_rsqrt(cnt)
            if first:
                for c in range(DP // L):
                    sl = pl.ds(c * L, L)
                    a_v[r, sl] = d * y_v[r, sl]
            else:
                t = d * d
                u = t * d
                for c in range(DP // L):
                    sl = pl.ds(c * L, L)
                    a_v[r, sl] = t * (a_v[r, sl] + b_v[r, sl]) + u * y_v[r, sl]
            return carry

        lax.fori_loop(0, SUB, row, 0)
        pltpu.sync_copy(a_v, gout_hbm.at[cid, pl.ds(base, SUB)])

    plsc.subcore_barrier()

    _edge_pipeline(acc, src_all, dst_all, rows, gsem, ssem, gout_hbm.at[cid])

    plsc.subcore_barrier()

    pltpu.sync_copy(acc.at[pl.ds(sid * RPS, RPS)],
                    parts_hbm.at[cid, pl.ds(sid * RPS, RPS)])


_hop_ring_scratch = (
    [pltpu.VMEM((B, DP), jnp.float32)] * NBUF
    + [pltpu.SemaphoreType.DMA] * (2 * NBUF))

_hop1 = pl.kernel(
    _hop1_body,
    out_type=(jax.ShapeDtypeStruct((NC, NP, DP), jnp.float32),
              jax.ShapeDtypeStruct((NC, NP, DP), jnp.float32)),
    mesh=_MESH,
    scratch_types=[
        pltpu.VMEM_SHARED((NP, DP), jnp.float32),
        pltpu.VMEM((NCH, B), jnp.int32),
        pltpu.VMEM((NCH, B), jnp.int32),
        pltpu.VMEM((SUB, DP), jnp.float32),
        pltpu.VMEM((SUB, DP), jnp.float32),
        pltpu.VMEM((SUB, DEGW), jnp.float32),
        pltpu.VMEM((SUB, DEGW), jnp.float32),
    ] + _hop_ring_scratch,
    compiler_params=_SC_PARAMS,
)

_hop2 = pl.kernel(
    _hop2_body,
    out_type=(jax.ShapeDtypeStruct((NC, NP, DP), jnp.float32),
              jax.ShapeDtypeStruct((NC, NP, DP), jnp.float32)),
    mesh=_MESH,
    scratch_types=[
        pltpu.VMEM_SHARED((NP, DP), jnp.float32),
        pltpu.VMEM((NCH, B), jnp.int32),
        pltpu.VMEM((NCH, B), jnp.int32),
        pltpu.VMEM((SUB, DP), jnp.float32),
        pltpu.VMEM((SUB, DP), jnp.float32),
        pltpu.VMEM((SUB, DP), jnp.float32),
        pltpu.VMEM((SUB, DEGW), jnp.float32),
        pltpu.VMEM((SUB, DEGW), jnp.float32),
    ] + _hop_ring_scratch,
    compiler_params=_SC_PARAMS,
)


def _deg_body(ones_hbm, dst_hbm, z_hbm, out_hbm, acc, dst_all, ones_v, sem):
    cid = lax.axis_index("c")
    sid = lax.axis_index("s")
    w = cid * NS + sid

    pltpu.sync_copy(z_hbm.at[pl.ds(sid * RPS, RPS)],
                    acc.at[pl.ds(sid * RPS, RPS)])
    pltpu.sync_copy(dst_hbm.at[w], dst_all)
    pltpu.sync_copy(ones_hbm, ones_v)
    plsc.subcore_barrier()

    def issue(j):
        pltpu.async_copy(ones_v, acc.at[dst_all.at[j]], sem, add=True)

    def drain_one():
        pltpu.make_async_copy(ones_v, acc.at[dst_all.at[0]], sem).wait()

    for j in range(DEG_LAG):
        issue(j)

    def step(j, carry):
        issue(j)
        drain_one()
        return carry

    lax.fori_loop(DEG_LAG, NCH, step, 0)
    for _ in range(DEG_LAG):
        drain_one()

    plsc.subcore_barrier()
    pltpu.sync_copy(acc.at[pl.ds(sid * RPS, RPS)],
                    out_hbm.at[cid, pl.ds(sid * RPS, RPS)])


_deg = pl.kernel(
    _deg_body,
    out_type=jax.ShapeDtypeStruct((NC, NP, DEGW), jnp.float32),
    mesh=_MESH,
    scratch_types=[
        pltpu.VMEM_SHARED((NP, DEGW), jnp.float32),
        pltpu.VMEM((NCH, B), jnp.int32),
        pltpu.VMEM((B, DEGW), jnp.float32),
        pltpu.SemaphoreType.DMA,
    ],
    compiler_params=_SC_PARAMS,
)


def _finish_body(dp_hbm, pp_hbm, g_hbm, out_hbm,
                 dp0, dp1, p0_v, p1_v, g_v, o_v):
    cid = lax.axis_index("c")
    sid = lax.axis_index("s")
    w = cid * NS + sid
    base = w * RPW

    pltpu.sync_copy(dp_hbm.at[0, pl.ds(base, RPW)], dp0)
    pltpu.sync_copy(dp_hbm.at[1, pl.ds(base, RPW)], dp1)
    pltpu.sync_copy(pp_hbm.at[0, pl.ds(base, RPW)], p0_v)
    pltpu.sync_copy(pp_hbm.at[1, pl.ds(base, RPW)], p1_v)
    pltpu.sync_copy(g_hbm.at[0, pl.ds(base, RPW)], g_v)

    def row(r, carry):
        cnt = dp0[r, :] + dp1[r, :] + 1.0
        d = _fast_rsqrt(cnt)
        for c in range(DP // L):
            sl = pl.ds(c * L, L)
            o_v[r, sl] = d * (p0_v[r, sl] + p1_v[r, sl] + g_v[r, sl])
        return carry

    lax.fori_loop(0, RPW, row, 0)
    pltpu.sync_copy(o_v, out_hbm.at[pl.ds(base, RPW)])


_finish = pl.kernel(
    _finish_body,
    out_type=jax.ShapeDtypeStruct((NP, DP), jnp.float32),
    mesh=_MESH,
    scratch_types=[
        pltpu.VMEM((RPW, DEGW), jnp.float32),
        pltpu.VMEM((RPW, DEGW), jnp.float32),
        pltpu.VMEM((RPW, DP), jnp.float32),
        pltpu.VMEM((RPW, DP), jnp.float32),
        pltpu.VMEM((RPW, DP), jnp.float32),
        pltpu.VMEM((RPW, DP), jnp.float32),
    ],
    compiler_params=_SC_PARAMS,
)


def _mm_body(x_ref, w_ref, y_ref):
    y_ref[...] = jnp.dot(x_ref[...], w_ref[...],
                         preferred_element_type=jnp.float32)


_mm = pl.pallas_call(
    _mm_body,
    out_shape=jax.ShapeDtypeStruct((NP, DP), jnp.float32),
)


@jax.jit
def kernel(x, edge_index, W):
    src_r = edge_index[0].astype(jnp.int32).reshape(NW, NCH, B)
    dst_r = edge_index[1].astype(jnp.int32).reshape(NW, NCH, B)

    xp = jnp.pad(x, ((0, NP - N), (0, 0)))
    Wp = jnp.pad(W, ((0, 0), (0, DP - C)))
    zeros = jnp.zeros((NP, DP), jnp.float32)
    zeros16 = jnp.zeros((NP, DEGW), jnp.float32)
    ones16 = jnp.ones((B, DEGW), jnp.float32)

    deg_parts = _deg(ones16, dst_r, zeros16)
    y = _mm(xp, Wp)
    parts1, _ = _hop1(deg_parts, y, src_r, dst_r, zeros)
    parts2, g2r = _hop2(deg_parts, y, parts1, src_r, dst_r, zeros)
    outp = _finish(deg_parts, parts2, g2r)
    return outp[:N, :C]


# speedup vs baseline: 1.0192x; 1.0192x over previous
import functools

import jax
import jax.numpy as jnp
from jax import lax
from jax.experimental import pallas as pl
from jax.experimental.pallas import tpu as pltpu
from jax.experimental.pallas import tpu_sc as plsc

N = 10000
E = 320000
D = 128
C = 40
DP = 48
NP = 10240
NC = 2
NS = 16
NW = NC * NS
L = 16
B = 80
NCH = 125
RPS = NP // NS
RPW = NP // NW
SUB = 160
NBUF = 8
K = NBUF // 2
DEG_LAG = 8
DEGW = 16

_MESH = plsc.VectorSubcoreMesh(core_axis_name="c", subcore_axis_name="s")
_SC_PARAMS = pltpu.CompilerParams(use_tc_tiling_on_sc=False,
                                  needs_layout_passes=False)


def _fast_rsqrt(x):
    i = plsc.bitcast(x, jnp.int32)
    i = jnp.full((L,), 0x5F3759DF, jnp.int32) - lax.shift_right_logical(
        i, jnp.full((L,), 1, jnp.int32))
    r = plsc.bitcast(i, jnp.float32)
    r = r * (1.5 - 0.5 * x * r * r)
    r = r * (1.5 - 0.5 * x * r * r)
    return r

UNR = 4


def _edge_pipeline(acc, src_all, dst_all, rows, gsem, ssem, table):
    def gissue(b, j):
        pltpu.async_copy(table.at[src_all.at[j]], rows[b], gsem[b])

    def gwait(b):
        pltpu.make_async_copy(table.at[src_all.at[0]], rows[b],
                              gsem[b]).wait()

    def sissue(b, j):
        pltpu.async_copy(rows[b], acc.at[dst_all.at[j]], ssem[b], add=True)

    def swait(b):
        pltpu.make_async_copy(rows[b], acc.at[dst_all.at[0]], ssem[b]).wait()

    for j in range(K):
        gissue(j % NBUF, j)
    for j in range(K):
        gwait(j % NBUF)
        sissue(j % NBUF, j)
        gissue((j + K) % NBUF, j + K)

    n_grp = (NCH - 2 * K) // NBUF

    def grp(g, carry):
        for bi in range(NBUF):
            j = K + g * NBUF + bi
            b = (K + bi) % NBUF
            gwait(b)
            sissue(b, j)
            swait(bi)
            gissue(bi, j + K)
        return carry

    lax.fori_loop(0, n_grp, grp, 0)

    for j in range(K + n_grp * NBUF, NCH - K):
        gwait(j % NBUF)
        sissue(j % NBUF, j)
        swait((j - K) % NBUF)
        gissue((j + K) % NBUF, j + K)
    for j in range(NCH - K, NCH):
        gwait(j % NBUF)
        sissue(j % NBUF, j)
        swait((j - K) % NBUF)
    for j in range(NCH - K, NCH):
        swait(j % NBUF)


def _hop1_body(dp_hbm, y_hbm, src_hbm, dst_hbm, z_hbm, parts_hbm, gout_hbm,
               acc, src_all, dst_all, a_v, y_v, dp0, dp1, *ring):
    _hop_common(True, dp_hbm, y_hbm, None, src_hbm, dst_hbm, z_hbm,
                parts_hbm, gout_hbm, acc, src_all, dst_all,
                a_v, y_v, None, dp0, dp1, ring)


def _hop2_body(dp_hbm, y_hbm, pp_hbm, src_hbm, dst_hbm, z_hbm, parts_hbm,
               gout_hbm, acc, src_all, dst_all, a_v, y_v, b_v, dp0, dp1,
               *ring):
    _hop_common(False, dp_hbm, y_hbm, pp_hbm, src_hbm, dst_hbm, z_hbm,
                parts_hbm, gout_hbm, acc, src_all, dst_all,
                a_v, y_v, b_v, dp0, dp1, ring)


def _hop_common(first, dp_hbm, y_hbm, pp_hbm, src_hbm, dst_hbm, z_hbm,
                parts_hbm, gout_hbm, acc, src_all, dst_all,
                a_v, y_v, b_v, dp0, dp1, ring):
    cid = lax.axis_index("c")
    sid = lax.axis_index("s")
    w = cid * NS + sid
    rows = ring[:NBUF]
    gsem = ring[NBUF:2 * NBUF]
    ssem = ring[2 * NBUF:]

    pltpu.sync_copy(z_hbm.at[pl.ds(sid * RPS, RPS)],
                    acc.at[pl.ds(sid * RPS, RPS)])
    pltpu.sync_copy(src_hbm.at[w], src_all)
    pltpu.sync_copy(dst_hbm.at[w], dst_all)

    for s in range(RPS // SUB):
        base = sid * RPS + s * SUB
        pltpu.sync_copy(dp_hbm.at[0, pl.ds(base, SUB)], dp0)
        pltpu.sync_copy(dp_hbm.at[1, pl.ds(base, SUB)], dp1)
        pltpu.sync_copy(y_hbm.at[pl.ds(base, SUB)], y_v)
        if not first:
            pltpu.sync_copy(pp_hbm.at[0, pl.ds(base, SUB)], a_v)
            pltpu.sync_copy(pp_hbm.at[1, pl.ds(base, SUB)], b_v)

        def row(r4, carry):
            for k in range(UNR):
                r = r4 * UNR + k
                cnt = dp0[r, :] + dp1[r, :] + 1.0
                d = _fast_rsqrt(cnt)
                if first:
                    for c in range(DP // L):
                        sl = pl.ds(c * L, L)
                        a_v[r, sl] = d * y_v[r, sl]
                else:
                    t = d * d
                    u = t * d
                    for c in range(DP // L):
                        sl = pl.ds(c * L, L)
                        a_v[r, sl] = (t * (a_v[r, sl] + b_v[r, sl])
                                      + u * y_v[r, sl])
            return carry

        lax.fori_loop(0, SUB // UNR, row, 0)
        pltpu.sync_copy(a_v, gout_hbm.at[cid, pl.ds(base, SUB)])

    plsc.subcore_barrier()

    _edge_pipeline(acc, src_all, dst_all, rows, gsem, ssem, gout_hbm.at[cid])

    plsc.subcore_barrier()

    pltpu.sync_copy(acc.at[pl.ds(sid * RPS, RPS)],
                    parts_hbm.at[cid, pl.ds(sid * RPS, RPS)])


_hop_ring_scratch = (
    [pltpu.VMEM((B, DP), jnp.float32)] * NBUF
    + [pltpu.SemaphoreType.DMA] * (2 * NBUF))

_hop1 = pl.kernel(
    _hop1_body,
    out_type=(jax.ShapeDtypeStruct((NC, NP, DP), jnp.float32),
              jax.ShapeDtypeStruct((NC, NP, DP), jnp.float32)),
    mesh=_MESH,
    scratch_types=[
        pltpu.VMEM_SHARED((NP, DP), jnp.float32),
        pltpu.VMEM((NCH, B), jnp.int32),
        pltpu.VMEM((NCH, B), jnp.int32),
        pltpu.VMEM((SUB, DP), jnp.float32),
        pltpu.VMEM((SUB, DP), jnp.float32),
        pltpu.VMEM((SUB, DEGW), jnp.float32),
        pltpu.VMEM((SUB, DEGW), jnp.float32),
    ] + _hop_ring_scratch,
    compiler_params=_SC_PARAMS,
)

_hop2 = pl.kernel(
    _hop2_body,
    out_type=(jax.ShapeDtypeStruct((NC, NP, DP), jnp.float32),
              jax.ShapeDtypeStruct((NC, NP, DP), jnp.float32)),
    mesh=_MESH,
    scratch_types=[
        pltpu.VMEM_SHARED((NP, DP), jnp.float32),
        pltpu.VMEM((NCH, B), jnp.int32),
        pltpu.VMEM((NCH, B), jnp.int32),
        pltpu.VMEM((SUB, DP), jnp.float32),
        pltpu.VMEM((SUB, DP), jnp.float32),
        pltpu.VMEM((SUB, DP), jnp.float32),
        pltpu.VMEM((SUB, DEGW), jnp.float32),
        pltpu.VMEM((SUB, DEGW), jnp.float32),
    ] + _hop_ring_scratch,
    compiler_params=_SC_PARAMS,
)


def _deg_body(ones_hbm, dst_hbm, z_hbm, out_hbm, acc, dst_all, ones_v, sem):
    cid = lax.axis_index("c")
    sid = lax.axis_index("s")
    w = cid * NS + sid

    pltpu.sync_copy(z_hbm.at[pl.ds(sid * RPS, RPS)],
                    acc.at[pl.ds(sid * RPS, RPS)])
    pltpu.sync_copy(dst_hbm.at[w], dst_all)
    pltpu.sync_copy(ones_hbm, ones_v)
    plsc.subcore_barrier()

    def issue(j):
        pltpu.async_copy(ones_v, acc.at[dst_all.at[j]], sem, add=True)

    def drain_one():
        pltpu.make_async_copy(ones_v, acc.at[dst_all.at[0]], sem).wait()

    for j in range(DEG_LAG):
        issue(j)

    def step(j, carry):
        issue(j)
        drain_one()
        return carry

    lax.fori_loop(DEG_LAG, NCH, step, 0)
    for _ in range(DEG_LAG):
        drain_one()

    plsc.subcore_barrier()
    pltpu.sync_copy(acc.at[pl.ds(sid * RPS, RPS)],
                    out_hbm.at[cid, pl.ds(sid * RPS, RPS)])


_deg = pl.kernel(
    _deg_body,
    out_type=jax.ShapeDtypeStruct((NC, NP, DEGW), jnp.float32),
    mesh=_MESH,
    scratch_types=[
        pltpu.VMEM_SHARED((NP, DEGW), jnp.float32),
        pltpu.VMEM((NCH, B), jnp.int32),
        pltpu.VMEM((B, DEGW), jnp.float32),
        pltpu.SemaphoreType.DMA,
    ],
    compiler_params=_SC_PARAMS,
)


def _finish_body(dp_hbm, pp_hbm, g_hbm, out_hbm,
                 dp0, dp1, p0_v, p1_v, g_v, o_v):
    cid = lax.axis_index("c")
    sid = lax.axis_index("s")
    w = cid * NS + sid
    base = w * RPW

    pltpu.sync_copy(dp_hbm.at[0, pl.ds(base, RPW)], dp0)
    pltpu.sync_copy(dp_hbm.at[1, pl.ds(base, RPW)], dp1)
    pltpu.sync_copy(pp_hbm.at[0, pl.ds(base, RPW)], p0_v)
    pltpu.sync_copy(pp_hbm.at[1, pl.ds(base, RPW)], p1_v)
    pltpu.sync_copy(g_hbm.at[0, pl.ds(base, RPW)], g_v)

    def row(r4, carry):
        for k in range(UNR):
            r = r4 * UNR + k
            cnt = dp0[r, :] + dp1[r, :] + 1.0
            d = _fast_rsqrt(cnt)
            for c in range(DP // L):
                sl = pl.ds(c * L, L)
                o_v[r, sl] = d * (p0_v[r, sl] + p1_v[r, sl] + g_v[r, sl])
        return carry

    lax.fori_loop(0, RPW // UNR, row, 0)
    pltpu.sync_copy(o_v, out_hbm.at[pl.ds(base, RPW)])


_finish = pl.kernel(
    _finish_body,
    out_type=jax.ShapeDtypeStruct((NP, DP), jnp.float32),
    mesh=_MESH,
    scratch_types=[
        pltpu.VMEM((RPW, DEGW), jnp.float32),
        pltpu.VMEM((RPW, DEGW), jnp.float32),
        pltpu.VMEM((RPW, DP), jnp.float32),
        pltpu.VMEM((RPW, DP), jnp.float32),
        pltpu.VMEM((RPW, DP), jnp.float32),
        pltpu.VMEM((RPW, DP), jnp.float32),
    ],
    compiler_params=_SC_PARAMS,
)


def _mm_body(x_ref, w_ref, y_ref):
    y_ref[...] = jnp.dot(x_ref[...], w_ref[...],
                         preferred_element_type=jnp.float32)


_mm = pl.pallas_call(
    _mm_body,
    out_shape=jax.ShapeDtypeStruct((NP, DP), jnp.float32),
)


@jax.jit
def kernel(x, edge_index, W):
    src_r = edge_index[0].astype(jnp.int32).reshape(NW, NCH, B)
    dst_r = edge_index[1].astype(jnp.int32).reshape(NW, NCH, B)

    xp = jnp.pad(x, ((0, NP - N), (0, 0)))
    Wp = jnp.pad(W, ((0, 0), (0, DP - C)))
    zeros = jnp.zeros((NP, DP), jnp.float32)
    zeros16 = jnp.zeros((NP, DEGW), jnp.float32)
    ones16 = jnp.ones((B, DEGW), jnp.float32)

    deg_parts = _deg(ones16, dst_r, zeros16)
    y = _mm(xp, Wp)
    parts1, _ = _hop1(deg_parts, y, src_r, dst_r, zeros)
    parts2, g2r = _hop2(deg_parts, y, parts1, src_r, dst_r, zeros)
    outp = _finish(deg_parts, parts2, g2r)
    return outp[:N, :C]


# speedup vs baseline: 1.1727x; 1.1507x over previous
import functools

import jax
import jax.numpy as jnp
from jax import lax
from jax.experimental import pallas as pl
from jax.experimental.pallas import tpu as pltpu
from jax.experimental.pallas import tpu_sc as plsc

N = 10000
E = 320000
D = 128
C = 40
DP = 48
NP = 10240
NC = 2
NS = 16
NW = NC * NS
L = 16
B = 80
NCH = 125
RPS = NP // NS
RPW = NP // NW
SUB = 128
NBUF = 8
K = NBUF // 2
DEG_LAG = 8
DEGW = 16

_MESH = plsc.VectorSubcoreMesh(core_axis_name="c", subcore_axis_name="s")
_SC_PARAMS = pltpu.CompilerParams(use_tc_tiling_on_sc=False,
                                  needs_layout_passes=False)


def _fast_rsqrt(x):
    i = plsc.bitcast(x, jnp.int32)
    i = jnp.full((L,), 0x5F3759DF, jnp.int32) - lax.shift_right_logical(
        i, jnp.full((L,), 1, jnp.int32))
    r = plsc.bitcast(i, jnp.float32)
    r = r * (1.5 - 0.5 * x * r * r)
    r = r * (1.5 - 0.5 * x * r * r)
    return r

UNR = 4


def _edge_pipeline(acc, src_all, dst_all, rows, gsem, ssem, table):
    def gissue(b, j):
        pltpu.async_copy(table.at[src_all.at[j]], rows[b], gsem[b])

    def gwait(b):
        pltpu.make_async_copy(table.at[src_all.at[0]], rows[b],
                              gsem[b]).wait()

    def sissue(b, j):
        pltpu.async_copy(rows[b], acc.at[dst_all.at[j]], ssem[b], add=True)

    def swait(b):
        pltpu.make_async_copy(rows[b], acc.at[dst_all.at[0]], ssem[b]).wait()

    for j in range(K):
        gissue(j % NBUF, j)
    for j in range(K):
        gwait(j % NBUF)
        sissue(j % NBUF, j)
        gissue((j + K) % NBUF, j + K)

    n_grp = (NCH - 2 * K) // NBUF

    def grp(g, carry):
        for bi in range(NBUF):
            j = K + g * NBUF + bi
            b = (K + bi) % NBUF
            gwait(b)
            sissue(b, j)
            swait(bi)
            gissue(bi, j + K)
        return carry

    lax.fori_loop(0, n_grp, grp, 0)

    for j in range(K + n_grp * NBUF, NCH - K):
        gwait(j % NBUF)
        sissue(j % NBUF, j)
        swait((j - K) % NBUF)
        gissue((j + K) % NBUF, j + K)
    for j in range(NCH - K, NCH):
        gwait(j % NBUF)
        sissue(j % NBUF, j)
        swait((j - K) % NBUF)
    for j in range(NCH - K, NCH):
        swait(j % NBUF)


def _hop1_body(dp_hbm, y_hbm, src_hbm, dst_hbm, z_hbm, parts_hbm, gout_hbm,
               acc, src_all, dst_all, a_v, y_v, dp0, dp1, misc_sem,
               ld_sem, st_sem, *ring):
    _hop_common(True, dp_hbm, y_hbm, None, src_hbm, dst_hbm, z_hbm,
                parts_hbm, gout_hbm, acc, src_all, dst_all,
                a_v, y_v, None, dp0, dp1, misc_sem, ld_sem, st_sem, ring)


def _hop2_body(dp_hbm, y_hbm, pp_hbm, src_hbm, dst_hbm, z_hbm, parts_hbm,
               gout_hbm, acc, src_all, dst_all, a_v, y_v, b_v, dp0, dp1,
               misc_sem, ld_sem, st_sem, *ring):
    _hop_common(False, dp_hbm, y_hbm, pp_hbm, src_hbm, dst_hbm, z_hbm,
                parts_hbm, gout_hbm, acc, src_all, dst_all,
                a_v, y_v, b_v, dp0, dp1, misc_sem, ld_sem, st_sem, ring)


def _hop_common(first, dp_hbm, y_hbm, pp_hbm, src_hbm, dst_hbm, z_hbm,
                parts_hbm, gout_hbm, acc, src_all, dst_all,
                a_v, y_v, b_v, dp0, dp1, misc_sem, ld_sem, st_sem, ring):
    cid = lax.axis_index("c")
    sid = lax.axis_index("s")
    w = cid * NS + sid
    rows = ring[:NBUF]
    gsem = ring[NBUF:2 * NBUF]
    ssem = ring[2 * NBUF:]
    nsub = RPS // SUB

    pltpu.async_copy(z_hbm.at[pl.ds(sid * RPS, RPS)],
                     acc.at[pl.ds(sid * RPS, RPS)], misc_sem)
    pltpu.async_copy(src_hbm.at[w], src_all, misc_sem)
    pltpu.async_copy(dst_hbm.at[w], dst_all, misc_sem)

    def sub_base(s):
        return sid * RPS + s * SUB

    def loads(s, p):
        base = sub_base(s)
        pltpu.async_copy(dp_hbm.at[0, pl.ds(base, SUB)], dp0[p], ld_sem[p])
        pltpu.async_copy(dp_hbm.at[1, pl.ds(base, SUB)], dp1[p], ld_sem[p])
        pltpu.async_copy(y_hbm.at[pl.ds(base, SUB)], y_v[p], ld_sem[p])
        if not first:
            pltpu.async_copy(pp_hbm.at[0, pl.ds(base, SUB)], a_v[p],
                             ld_sem[p])
            pltpu.async_copy(pp_hbm.at[1, pl.ds(base, SUB)], b_v[p],
                             ld_sem[p])

    def wait_loads(p):
        pltpu.make_async_copy(dp_hbm.at[0, pl.ds(0, SUB)], dp0[p],
                              ld_sem[p]).wait()
        pltpu.make_async_copy(dp_hbm.at[1, pl.ds(0, SUB)], dp1[p],
                              ld_sem[p]).wait()
        pltpu.make_async_copy(y_hbm.at[pl.ds(0, SUB)], y_v[p],
                              ld_sem[p]).wait()
        if not first:
            pltpu.make_async_copy(pp_hbm.at[0, pl.ds(0, SUB)], a_v[p],
                                  ld_sem[p]).wait()
            pltpu.make_async_copy(pp_hbm.at[1, pl.ds(0, SUB)], b_v[p],
                                  ld_sem[p]).wait()

    def wait_store(p):
        pltpu.make_async_copy(a_v[p], gout_hbm.at[cid, pl.ds(0, SUB)],
                              st_sem[p]).wait()

    loads(0, 0)
    loads(1, 1)
    for s in range(nsub):
        p = s % 2
        wait_loads(p)
        if s >= 2:
            wait_store(p)

        def row(r4, carry):
            for k in range(UNR):
                r = r4 * UNR + k
                cnt = dp0[p][r, :] + dp1[p][r, :] + 1.0
                d = _fast_rsqrt(cnt)
                if first:
                    for c in range(DP // L):
                        sl = pl.ds(c * L, L)
                        a_v[p][r, sl] = d * y_v[p][r, sl]
                else:
                    t = d * d
                    u = t * d
                    for c in range(DP // L):
                        sl = pl.ds(c * L, L)
                        a_v[p][r, sl] = (t * (a_v[p][r, sl] + b_v[p][r, sl])
                                         + u * y_v[p][r, sl])
            return carry

        lax.fori_loop(0, SUB // UNR, row, 0)
        pltpu.async_copy(a_v[p], gout_hbm.at[cid, pl.ds(sub_base(s), SUB)],
                         st_sem[p])
        if s + 2 < nsub:
            loads(s + 2, p)
    wait_store(nsub % 2)
    wait_store((nsub + 1) % 2)

    pltpu.make_async_copy(z_hbm.at[pl.ds(0, RPS)],
                          acc.at[pl.ds(0, RPS)], misc_sem).wait()
    pltpu.make_async_copy(src_hbm.at[0], src_all, misc_sem).wait()
    pltpu.make_async_copy(dst_hbm.at[0], dst_all, misc_sem).wait()

    plsc.subcore_barrier()

    _edge_pipeline(acc, src_all, dst_all, rows, gsem, ssem, gout_hbm.at[cid])

    plsc.subcore_barrier()

    pltpu.sync_copy(acc.at[pl.ds(sid * RPS, RPS)],
                    parts_hbm.at[cid, pl.ds(sid * RPS, RPS)])


_hop_ring_scratch = (
    [pltpu.VMEM((B, DP), jnp.float32)] * NBUF
    + [pltpu.SemaphoreType.DMA] * (2 * NBUF))

def _pair(shape, dtype):
    return (pltpu.VMEM(shape, dtype), pltpu.VMEM(shape, dtype))


_SEM_PAIR = (pltpu.SemaphoreType.DMA, pltpu.SemaphoreType.DMA)

_hop1 = pl.kernel(
    _hop1_body,
    out_type=(jax.ShapeDtypeStruct((NC, NP, DP), jnp.float32),
              jax.ShapeDtypeStruct((NC, NP, DP), jnp.float32)),
    mesh=_MESH,
    scratch_types=[
        pltpu.VMEM_SHARED((NP, DP), jnp.float32),
        pltpu.VMEM((NCH, B), jnp.int32),
        pltpu.VMEM((NCH, B), jnp.int32),
        _pair((SUB, DP), jnp.float32),
        _pair((SUB, DP), jnp.float32),
        _pair((SUB, DEGW), jnp.float32),
        _pair((SUB, DEGW), jnp.float32),
        pltpu.SemaphoreType.DMA,
        _SEM_PAIR,
        _SEM_PAIR,
    ] + _hop_ring_scratch,
    compiler_params=_SC_PARAMS,
)

_hop2 = pl.kernel(
    _hop2_body,
    out_type=(jax.ShapeDtypeStruct((NC, NP, DP), jnp.float32),
              jax.ShapeDtypeStruct((NC, NP, DP), jnp.float32)),
    mesh=_MESH,
    scratch_types=[
        pltpu.VMEM_SHARED((NP, DP), jnp.float32),
        pltpu.VMEM((NCH, B), jnp.int32),
        pltpu.VMEM((NCH, B), jnp.int32),
        _pair((SUB, DP), jnp.float32),
        _pair((SUB, DP), jnp.float32),
        _pair((SUB, DP), jnp.float32),
        _pair((SUB, DEGW), jnp.float32),
        _pair((SUB, DEGW), jnp.float32),
        pltpu.SemaphoreType.DMA,
        _SEM_PAIR,
        _SEM_PAIR,
    ] + _hop_ring_scratch,
    compiler_params=_SC_PARAMS,
)


def _deg_body(ones_hbm, dst_hbm, z_hbm, out_hbm, acc, dst_all, ones_v, sem):
    cid = lax.axis_index("c")
    sid = lax.axis_index("s")
    w = cid * NS + sid

    pltpu.sync_copy(z_hbm.at[pl.ds(sid * RPS, RPS)],
                    acc.at[pl.ds(sid * RPS, RPS)])
    pltpu.sync_copy(dst_hbm.at[w], dst_all)
    pltpu.sync_copy(ones_hbm, ones_v)
    plsc.subcore_barrier()

    def issue(j):
        pltpu.async_copy(ones_v, acc.at[dst_all.at[j]], sem, add=True)

    def drain_one():
        pltpu.make_async_copy(ones_v, acc.at[dst_all.at[0]], sem).wait()

    for j in range(DEG_LAG):
        issue(j)

    def step(j, carry):
        issue(j)
        drain_one()
        return carry

    lax.fori_loop(DEG_LAG, NCH, step, 0)
    for _ in range(DEG_LAG):
        drain_one()

    plsc.subcore_barrier()
    pltpu.sync_copy(acc.at[pl.ds(sid * RPS, RPS)],
                    out_hbm.at[cid, pl.ds(sid * RPS, RPS)])


_deg = pl.kernel(
    _deg_body,
    out_type=jax.ShapeDtypeStruct((NC, NP, DEGW), jnp.float32),
    mesh=_MESH,
    scratch_types=[
        pltpu.VMEM_SHARED((NP, DEGW), jnp.float32),
        pltpu.VMEM((NCH, B), jnp.int32),
        pltpu.VMEM((B, DEGW), jnp.float32),
        pltpu.SemaphoreType.DMA,
    ],
    compiler_params=_SC_PARAMS,
)


def _finish_body(dp_hbm, pp_hbm, g_hbm, out_hbm,
                 dp0, dp1, p0_v, p1_v, g_v, o_v, sem):
    cid = lax.axis_index("c")
    sid = lax.axis_index("s")
    w = cid * NS + sid
    base = w * RPW

    pairs = [(dp_hbm.at[0, pl.ds(base, RPW)], dp0),
             (dp_hbm.at[1, pl.ds(base, RPW)], dp1),
             (pp_hbm.at[0, pl.ds(base, RPW)], p0_v),
             (pp_hbm.at[1, pl.ds(base, RPW)], p1_v),
             (g_hbm.at[0, pl.ds(base, RPW)], g_v)]
    for s, d in pairs:
        pltpu.async_copy(s, d, sem)
    for s, d in pairs:
        pltpu.make_async_copy(s, d, sem).wait()

    def row(r4, carry):
        for k in range(UNR):
            r = r4 * UNR + k
            cnt = dp0[r, :] + dp1[r, :] + 1.0
            d = _fast_rsqrt(cnt)
            for c in range(DP // L):
                sl = pl.ds(c * L, L)
                o_v[r, sl] = d * (p0_v[r, sl] + p1_v[r, sl] + g_v[r, sl])
        return carry

    lax.fori_loop(0, RPW // UNR, row, 0)
    pltpu.sync_copy(o_v, out_hbm.at[pl.ds(base, RPW)])


_finish = pl.kernel(
    _finish_body,
    out_type=jax.ShapeDtypeStruct((NP, DP), jnp.float32),
    mesh=_MESH,
    scratch_types=[
        pltpu.VMEM((RPW, DEGW), jnp.float32),
        pltpu.VMEM((RPW, DEGW), jnp.float32),
        pltpu.VMEM((RPW, DP), jnp.float32),
        pltpu.VMEM((RPW, DP), jnp.float32),
        pltpu.VMEM((RPW, DP), jnp.float32),
        pltpu.VMEM((RPW, DP), jnp.float32),
        pltpu.SemaphoreType.DMA,
    ],
    compiler_params=_SC_PARAMS,
)


def _mm_body(x_ref, w_ref, y_ref):
    y_ref[...] = jnp.dot(x_ref[...], w_ref[...],
                         preferred_element_type=jnp.float32)


_mm = pl.pallas_call(
    _mm_body,
    out_shape=jax.ShapeDtypeStruct((NP, DP), jnp.float32),
)


@jax.jit
def kernel(x, edge_index, W):
    src_r = edge_index[0].astype(jnp.int32).reshape(NW, NCH, B)
    dst_r = edge_index[1].astype(jnp.int32).reshape(NW, NCH, B)

    xp = jnp.pad(x, ((0, NP - N), (0, 0)))
    Wp = jnp.pad(W, ((0, 0), (0, DP - C)))
    zeros = jnp.zeros((NP, DP), jnp.float32)
    zeros16 = jnp.zeros((NP, DEGW), jnp.float32)
    ones16 = jnp.ones((B, DEGW), jnp.float32)

    deg_parts = _deg(ones16, dst_r, zeros16)
    y = _mm(xp, Wp)
    parts1, _ = _hop1(deg_parts, y, src_r, dst_r, zeros)
    parts2, g2r = _hop2(deg_parts, y, parts1, src_r, dst_r, zeros)
    outp = _finish(deg_parts, parts2, g2r)
    return outp[:N, :C]


# speedup vs baseline: 1.2605x; 1.0749x over previous
import functools

import jax
import jax.numpy as jnp
from jax import lax
from jax.experimental import pallas as pl
from jax.experimental.pallas import tpu as pltpu
from jax.experimental.pallas import tpu_sc as plsc

N = 10000
E = 320000
D = 128
C = 40
DP = 48
NP = 10240
NC = 2
NS = 16
NW = NC * NS
L = 16
B = 128
NCH = 78
ECH = E // B
RPS = NP // NS
RPW = NP // NW
SUB = 64
NBUF = 8
K = NBUF // 2
DEG_LAG = 8
DEGW = 16

_MESH = plsc.VectorSubcoreMesh(core_axis_name="c", subcore_axis_name="s")
_SC_PARAMS = pltpu.CompilerParams(use_tc_tiling_on_sc=False,
                                  needs_layout_passes=False)


def _fast_rsqrt(x):
    i = plsc.bitcast(x, jnp.int32)
    i = jnp.full((L,), 0x5F3759DF, jnp.int32) - lax.shift_right_logical(
        i, jnp.full((L,), 1, jnp.int32))
    r = plsc.bitcast(i, jnp.float32)
    r = r * (1.5 - 0.5 * x * r * r)
    r = r * (1.5 - 0.5 * x * r * r)
    return r

UNR = 4


def _edge_pipeline(acc, src_all, dst_all, rows, gsem, ssem, table):
    def gissue(b, j):
        pltpu.async_copy(table.at[src_all.at[j]], rows[b], gsem[b])

    def gwait(b):
        pltpu.make_async_copy(table.at[src_all.at[0]], rows[b],
                              gsem[b]).wait()

    def sissue(b, j):
        pltpu.async_copy(rows[b], acc.at[dst_all.at[j]], ssem[b], add=True)

    def swait(b):
        pltpu.make_async_copy(rows[b], acc.at[dst_all.at[0]], ssem[b]).wait()

    for j in range(K):
        gissue(j % NBUF, j)
    for j in range(K):
        gwait(j % NBUF)
        sissue(j % NBUF, j)
        gissue((j + K) % NBUF, j + K)

    n_grp = (NCH - 2 * K) // NBUF

    def grp(g, carry):
        for bi in range(NBUF):
            j = K + g * NBUF + bi
            b = (K + bi) % NBUF
            gwait(b)
            sissue(b, j)
            swait(bi)
            gissue(bi, j + K)
        return carry

    lax.fori_loop(0, n_grp, grp, 0)

    for j in range(K + n_grp * NBUF, NCH - K):
        gwait(j % NBUF)
        sissue(j % NBUF, j)
        swait((j - K) % NBUF)
        gissue((j + K) % NBUF, j + K)
    for j in range(NCH - K, NCH):
        gwait(j % NBUF)
        sissue(j % NBUF, j)
        swait((j - K) % NBUF)
    for j in range(NCH - K, NCH):
        swait(j % NBUF)


def _hop1_body(dp_hbm, y_hbm, ei_hbm, parts_hbm, gout_hbm,
               acc, src_all, dst_all, xsrc, xdst, a_v, y_v, dp0, dp1,
               misc_sem, zsem, ld_sem, st_sem, *ring):
    _hop_common(True, dp_hbm, y_hbm, None, ei_hbm,
                parts_hbm, gout_hbm, acc, src_all, dst_all, xsrc, xdst,
                a_v, y_v, None, dp0, dp1, misc_sem, zsem, ld_sem, st_sem,
                ring)


def _hop2_body(dp_hbm, y_hbm, pp_hbm, ei_hbm, parts_hbm,
               gout_hbm, acc, src_all, dst_all, xsrc, xdst, a_v, y_v, b_v,
               dp0, dp1, misc_sem, zsem, ld_sem, st_sem, *ring):
    _hop_common(False, dp_hbm, y_hbm, pp_hbm, ei_hbm,
                parts_hbm, gout_hbm, acc, src_all, dst_all, xsrc, xdst,
                a_v, y_v, b_v, dp0, dp1, misc_sem, zsem, ld_sem, st_sem,
                ring)


def _zero_vmem(buf, nrow, ncolgrp):
    zv = jnp.zeros((L,), jnp.float32)

    def zrow(r, carry):
        for c in range(ncolgrp):
            buf[r, pl.ds(c * L, L)] = zv
        return carry

    lax.fori_loop(0, nrow, zrow, 0)


def _hop_common(first, dp_hbm, y_hbm, pp_hbm, ei_hbm,
                parts_hbm, gout_hbm, acc, src_all, dst_all, xsrc, xdst,
                a_v, y_v, b_v, dp0, dp1, misc_sem, zsem, ld_sem, st_sem,
                ring):
    cid = lax.axis_index("c")
    sid = lax.axis_index("s")
    w = cid * NS + sid
    rows = ring[:NBUF]
    gsem = ring[NBUF:2 * NBUF]
    ssem = ring[2 * NBUF:]
    nsub = RPS // SUB

    _zero_vmem(rows[0], B, DP // L)
    for q in range(RPS // B):
        pltpu.async_copy(rows[0], acc.at[pl.ds(sid * RPS + q * B, B)],
                         zsem)
    pltpu.async_copy(ei_hbm.at[0, pl.ds(w * NCH, NCH)], src_all, misc_sem)
    pltpu.async_copy(ei_hbm.at[1, pl.ds(w * NCH, NCH)], dst_all, misc_sem)

    def sub_base(s):
        return sid * RPS + s * SUB

    def loads(s, p):
        base = sub_base(s)
        pltpu.async_copy(dp_hbm.at[0, pl.ds(base, SUB)], dp0[p], ld_sem[p])
        pltpu.async_copy(dp_hbm.at[1, pl.ds(base, SUB)], dp1[p], ld_sem[p])
        pltpu.async_copy(y_hbm.at[pl.ds(base, SUB)], y_v[p], ld_sem[p])
        if not first:
            pltpu.async_copy(pp_hbm.at[0, pl.ds(base, SUB)], a_v[p],
                             ld_sem[p])
            pltpu.async_copy(pp_hbm.at[1, pl.ds(base, SUB)], b_v[p],
                             ld_sem[p])

    def wait_loads(p):
        pltpu.make_async_copy(dp_hbm.at[0, pl.ds(0, SUB)], dp0[p],
                              ld_sem[p]).wait()
        pltpu.make_async_copy(dp_hbm.at[1, pl.ds(0, SUB)], dp1[p],
                              ld_sem[p]).wait()
        pltpu.make_async_copy(y_hbm.at[pl.ds(0, SUB)], y_v[p],
                              ld_sem[p]).wait()
        if not first:
            pltpu.make_async_copy(pp_hbm.at[0, pl.ds(0, SUB)], a_v[p],
                                  ld_sem[p]).wait()
            pltpu.make_async_copy(pp_hbm.at[1, pl.ds(0, SUB)], b_v[p],
                                  ld_sem[p]).wait()

    def wait_store(p):
        pltpu.make_async_copy(a_v[p], gout_hbm.at[cid, pl.ds(0, SUB)],
                              st_sem[p]).wait()

    loads(0, 0)
    loads(1, 1)
    for s in range(nsub):
        p = s % 2
        wait_loads(p)
        if s >= 2:
            wait_store(p)

        def row(r4, carry):
            for k in range(UNR):
                r = r4 * UNR + k
                cnt = dp0[p][r, :] + dp1[p][r, :] + 1.0
                d = _fast_rsqrt(cnt)
                if first:
                    for c in range(DP // L):
                        sl = pl.ds(c * L, L)
                        a_v[p][r, sl] = d * y_v[p][r, sl]
                else:
                    t = d * d
                    u = t * d
                    for c in range(DP // L):
                        sl = pl.ds(c * L, L)
                        a_v[p][r, sl] = (t * (a_v[p][r, sl] + b_v[p][r, sl])
                                         + u * y_v[p][r, sl])
            return carry

        lax.fori_loop(0, SUB // UNR, row, 0)
        pltpu.async_copy(a_v[p], gout_hbm.at[cid, pl.ds(sub_base(s), SUB)],
                         st_sem[p])
        if s + 2 < nsub:
            loads(s + 2, p)
    wait_store(nsub % 2)
    wait_store((nsub + 1) % 2)

    for q in range(RPS // B):
        pltpu.make_async_copy(rows[0], acc.at[pl.ds(0, B)], zsem).wait()
    pltpu.make_async_copy(ei_hbm.at[0, pl.ds(0, NCH)], src_all,
                          misc_sem).wait()
    pltpu.make_async_copy(ei_hbm.at[1, pl.ds(0, NCH)], dst_all,
                          misc_sem).wait()

    plsc.subcore_barrier()

    _edge_pipeline(acc, src_all, dst_all, rows, gsem, ssem, gout_hbm.at[cid])

    @pl.when(w < ECH - NW * NCH)
    def _extra():
        pltpu.sync_copy(ei_hbm.at[0, NW * NCH + w], xsrc)
        pltpu.sync_copy(ei_hbm.at[1, NW * NCH + w], xdst)
        pltpu.async_copy(gout_hbm.at[cid].at[xsrc], rows[0], gsem[0])
        pltpu.make_async_copy(gout_hbm.at[cid].at[xsrc], rows[0],
                              gsem[0]).wait()
        pltpu.sync_copy(rows[0], acc.at[xdst], add=True)

    plsc.subcore_barrier()

    pltpu.sync_copy(acc.at[pl.ds(sid * RPS, RPS)],
                    parts_hbm.at[cid, pl.ds(sid * RPS, RPS)])


_hop_ring_scratch = (
    [pltpu.VMEM((B, DP), jnp.float32)] * NBUF
    + [pltpu.SemaphoreType.DMA] * (2 * NBUF))

def _pair(shape, dtype):
    return (pltpu.VMEM(shape, dtype), pltpu.VMEM(shape, dtype))


_SEM_PAIR = (pltpu.SemaphoreType.DMA, pltpu.SemaphoreType.DMA)

_hop1 = pl.kernel(
    _hop1_body,
    out_type=(jax.ShapeDtypeStruct((NC, NP, DP), jnp.float32),
              jax.ShapeDtypeStruct((NC, NP, DP), jnp.float32)),
    mesh=_MESH,
    scratch_types=[
        pltpu.VMEM_SHARED((NP, DP), jnp.float32),
        pltpu.VMEM((NCH, B), jnp.int32),
        pltpu.VMEM((NCH, B), jnp.int32),
        pltpu.VMEM((B,), jnp.int32),
        pltpu.VMEM((B,), jnp.int32),
        _pair((SUB, DP), jnp.float32),
        _pair((SUB, DP), jnp.float32),
        _pair((SUB, DEGW), jnp.float32),
        _pair((SUB, DEGW), jnp.float32),
        pltpu.SemaphoreType.DMA,
        pltpu.SemaphoreType.DMA,
        _SEM_PAIR,
        _SEM_PAIR,
    ] + _hop_ring_scratch,
    compiler_params=_SC_PARAMS,
)

_hop2 = pl.kernel(
    _hop2_body,
    out_type=(jax.ShapeDtypeStruct((NC, NP, DP), jnp.float32),
              jax.ShapeDtypeStruct((NC, NP, DP), jnp.float32)),
    mesh=_MESH,
    scratch_types=[
        pltpu.VMEM_SHARED((NP, DP), jnp.float32),
        pltpu.VMEM((NCH, B), jnp.int32),
        pltpu.VMEM((NCH, B), jnp.int32),
        pltpu.VMEM((B,), jnp.int32),
        pltpu.VMEM((B,), jnp.int32),
        _pair((SUB, DP), jnp.float32),
        _pair((SUB, DP), jnp.float32),
        _pair((SUB, DP), jnp.float32),
        _pair((SUB, DEGW), jnp.float32),
        _pair((SUB, DEGW), jnp.float32),
        pltpu.SemaphoreType.DMA,
        pltpu.SemaphoreType.DMA,
        _SEM_PAIR,
        _SEM_PAIR,
    ] + _hop_ring_scratch,
    compiler_params=_SC_PARAMS,
)


def _deg_body(ei_hbm, out_hbm, acc, dst_all, xdst, ones_v, sem, zsem):
    cid = lax.axis_index("c")
    sid = lax.axis_index("s")
    w = cid * NS + sid

    pltpu.async_copy(ei_hbm.at[1, pl.ds(w * NCH, NCH)], dst_all, sem)
    _zero_vmem(ones_v, B, DEGW // L)
    for q in range(RPS // B):
        pltpu.async_copy(ones_v, acc.at[pl.ds(sid * RPS + q * B, B)], zsem)
    for q in range(RPS // B):
        pltpu.make_async_copy(ones_v, acc.at[pl.ds(0, B)], zsem).wait()
    ov = jnp.full((L,), 1.0, jnp.float32)

    def orow(r, carry):
        ones_v[r, pl.ds(0, L)] = ov
        return carry

    lax.fori_loop(0, B, orow, 0)
    pltpu.make_async_copy(ei_hbm.at[1, pl.ds(0, NCH)], dst_all, sem).wait()
    plsc.subcore_barrier()

    def issue(j):
        pltpu.async_copy(ones_v, acc.at[dst_all.at[j]], sem, add=True)

    def drain_one():
        pltpu.make_async_copy(ones_v, acc.at[dst_all.at[0]], sem).wait()

    for j in range(DEG_LAG):
        issue(j)

    def step(j, carry):
        issue(j)
        drain_one()
        return carry

    lax.fori_loop(DEG_LAG, NCH, step, 0)
    for _ in range(DEG_LAG):
        drain_one()

    @pl.when(w < ECH - NW * NCH)
    def _extra():
        pltpu.sync_copy(ei_hbm.at[1, NW * NCH + w], xdst)
        pltpu.sync_copy(ones_v, acc.at[xdst], add=True)

    plsc.subcore_barrier()
    pltpu.sync_copy(acc.at[pl.ds(sid * RPS, RPS)],
                    out_hbm.at[cid, pl.ds(sid * RPS, RPS)])


_deg = pl.kernel(
    _deg_body,
    out_type=jax.ShapeDtypeStruct((NC, NP, DEGW), jnp.float32),
    mesh=_MESH,
    scratch_types=[
        pltpu.VMEM_SHARED((NP, DEGW), jnp.float32),
        pltpu.VMEM((NCH, B), jnp.int32),
        pltpu.VMEM((B,), jnp.int32),
        pltpu.VMEM((B, DEGW), jnp.float32),
        pltpu.SemaphoreType.DMA,
        pltpu.SemaphoreType.DMA,
    ],
    compiler_params=_SC_PARAMS,
)


def _finish_body(dp_hbm, pp_hbm, g_hbm, out_hbm,
                 dp0, dp1, p0_v, p1_v, g_v, o_v, sem):
    cid = lax.axis_index("c")
    sid = lax.axis_index("s")
    w = cid * NS + sid
    base = w * RPW

    pairs = [(dp_hbm.at[0, pl.ds(base, RPW)], dp0),
             (dp_hbm.at[1, pl.ds(base, RPW)], dp1),
             (pp_hbm.at[0, pl.ds(base, RPW)], p0_v),
             (pp_hbm.at[1, pl.ds(base, RPW)], p1_v),
             (g_hbm.at[0, pl.ds(base, RPW)], g_v)]
    for s, d in pairs:
        pltpu.async_copy(s, d, sem)
    for s, d in pairs:
        pltpu.make_async_copy(s, d, sem).wait()

    def row(r4, carry):
        for k in range(UNR):
            r = r4 * UNR + k
            cnt = dp0[r, :] + dp1[r, :] + 1.0
            d = _fast_rsqrt(cnt)
            for c in range(DP // L):
                sl = pl.ds(c * L, L)
                o_v[r, sl] = d * (p0_v[r, sl] + p1_v[r, sl] + g_v[r, sl])
        return carry

    lax.fori_loop(0, RPW // UNR, row, 0)
    pltpu.sync_copy(o_v, out_hbm.at[pl.ds(base, RPW)])


_finish = pl.kernel(
    _finish_body,
    out_type=jax.ShapeDtypeStruct((NP, DP), jnp.float32),
    mesh=_MESH,
    scratch_types=[
        pltpu.VMEM((RPW, DEGW), jnp.float32),
        pltpu.VMEM((RPW, DEGW), jnp.float32),
        pltpu.VMEM((RPW, DP), jnp.float32),
        pltpu.VMEM((RPW, DP), jnp.float32),
        pltpu.VMEM((RPW, DP), jnp.float32),
        pltpu.VMEM((RPW, DP), jnp.float32),
        pltpu.SemaphoreType.DMA,
    ],
    compiler_params=_SC_PARAMS,
)


def _mm_body(x_ref, w_ref, y_ref):
    y_ref[pl.ds(0, N), :] = jnp.dot(x_ref[...], w_ref[...],
                                    preferred_element_type=jnp.float32)


_mm = pl.pallas_call(
    _mm_body,
    out_shape=jax.ShapeDtypeStruct((NP, DP), jnp.float32),
)


@jax.jit
def kernel(x, edge_index, W):
    ei3 = edge_index.astype(jnp.int32).reshape(2, ECH, B)

    Wp = jnp.pad(W, ((0, 0), (0, DP - C)))

    deg_parts = _deg(ei3)
    y = _mm(x, Wp)
    parts1, _ = _hop1(deg_parts, y, ei3)
    parts2, g2r = _hop2(deg_parts, y, parts1, ei3)
    outp = _finish(deg_parts, parts2, g2r)
    return outp[:N, :C]


# speedup vs baseline: 1.2620x; 1.0012x over previous
import jax
import jax.numpy as jnp
from jax import lax
from jax.experimental import pallas as pl
from jax.experimental.pallas import tpu as pltpu
from jax.experimental.pallas import tpu_sc as plsc

N = 10000
E = 320000
D = 128
C = 40
DP = 48
NP = 10240
NC = 2
NS = 16
NW = NC * NS
L = 16
B = 128
NCH = 78
ECH = E // B
RPS = NP // NS
RPW = NP // NW
SUB = 64
NBUF = 8
K = NBUF // 2
DEG_LAG = 8
DEGW = 16

_MESH = plsc.VectorSubcoreMesh(core_axis_name="c", subcore_axis_name="s")
_SC_PARAMS = pltpu.CompilerParams(use_tc_tiling_on_sc=False,
                                  needs_layout_passes=False)


def _fast_rsqrt(x):
    i = plsc.bitcast(x, jnp.int32)
    i = jnp.full((L,), 0x5F3759DF, jnp.int32) - lax.shift_right_logical(
        i, jnp.full((L,), 1, jnp.int32))
    r = plsc.bitcast(i, jnp.float32)
    r = r * (1.5 - 0.5 * x * r * r)
    r = r * (1.5 - 0.5 * x * r * r)
    return r

UNR = 4


def _edge_pipeline(acc, src_all, dst_all, rows, gsem, ssem, table):
    def gissue(b, j):
        pltpu.async_copy(table.at[src_all.at[j]], rows[b], gsem[b])

    def gwait(b):
        pltpu.make_async_copy(table.at[src_all.at[0]], rows[b],
                              gsem[b]).wait()

    def sissue(b, j):
        pltpu.async_copy(rows[b], acc.at[dst_all.at[j]], ssem[b], add=True)

    def swait(b):
        pltpu.make_async_copy(rows[b], acc.at[dst_all.at[0]], ssem[b]).wait()

    for j in range(K):
        gissue(j % NBUF, j)
    for j in range(K):
        gwait(j % NBUF)
        sissue(j % NBUF, j)
        gissue((j + K) % NBUF, j + K)

    n_grp = (NCH - 2 * K) // NBUF

    def grp(g, carry):
        for bi in range(NBUF):
            j = K + g * NBUF + bi
            b = (K + bi) % NBUF
            gwait(b)
            sissue(b, j)
            swait(bi)
            gissue(bi, j + K)
        return carry

    lax.fori_loop(0, n_grp, grp, 0)

    for j in range(K + n_grp * NBUF, NCH - K):
        gwait(j % NBUF)
        sissue(j % NBUF, j)
        swait((j - K) % NBUF)
        gissue((j + K) % NBUF, j + K)
    for j in range(NCH - K, NCH):
        gwait(j % NBUF)
        sissue(j % NBUF, j)
        swait((j - K) % NBUF)
    for j in range(NCH - K, NCH):
        swait(j % NBUF)


def _hop1_body(dp_hbm, y_hbm, ei_hbm, parts_hbm, gout_hbm,
               acc, src_all, dst_all, xsrc, xdst, a_v, y_v, dp0, dp1,
               misc_sem, zsem, ld_sem, st_sem, *ring):
    _hop_common(True, dp_hbm, y_hbm, None, ei_hbm,
                parts_hbm, gout_hbm, acc, src_all, dst_all, xsrc, xdst,
                a_v, y_v, None, dp0, dp1, misc_sem, zsem, ld_sem, st_sem,
                ring)


def _hop2_body(dp_hbm, y_hbm, pp_hbm, ei_hbm, parts_hbm,
               gout_hbm, acc, src_all, dst_all, xsrc, xdst, a_v, y_v, b_v,
               dp0, dp1, misc_sem, zsem, ld_sem, st_sem, *ring):
    _hop_common(False, dp_hbm, y_hbm, pp_hbm, ei_hbm,
                parts_hbm, gout_hbm, acc, src_all, dst_all, xsrc, xdst,
                a_v, y_v, b_v, dp0, dp1, misc_sem, zsem, ld_sem, st_sem,
                ring)


def _zero_vmem(buf, nrow, ncolgrp):
    zv = jnp.zeros((L,), jnp.float32)

    def zrow(r, carry):
        for c in range(ncolgrp):
            buf[r, pl.ds(c * L, L)] = zv
        return carry

    lax.fori_loop(0, nrow, zrow, 0)


def _hop_common(first, dp_hbm, y_hbm, pp_hbm, ei_hbm,
                parts_hbm, gout_hbm, acc, src_all, dst_all, xsrc, xdst,
                a_v, y_v, b_v, dp0, dp1, misc_sem, zsem, ld_sem, st_sem,
                ring):
    cid = lax.axis_index("c")
    sid = lax.axis_index("s")
    w = cid * NS + sid
    rows = ring[:NBUF]
    gsem = ring[NBUF:2 * NBUF]
    ssem = ring[2 * NBUF:]
    nsub = RPS // SUB

    _zero_vmem(rows[0], B, DP // L)
    for q in range(RPS // B):
        pltpu.async_copy(rows[0], acc.at[pl.ds(sid * RPS + q * B, B)],
                         zsem)
    pltpu.async_copy(ei_hbm.at[0, pl.ds(w * NCH, NCH)], src_all, misc_sem)
    pltpu.async_copy(ei_hbm.at[1, pl.ds(w * NCH, NCH)], dst_all, misc_sem)

    def sub_base(s):
        return sid * RPS + s * SUB

    def loads(s, p):
        base = sub_base(s)
        pltpu.async_copy(dp_hbm.at[0, pl.ds(base, SUB)], dp0[p], ld_sem[p])
        pltpu.async_copy(dp_hbm.at[1, pl.ds(base, SUB)], dp1[p], ld_sem[p])
        pltpu.async_copy(y_hbm.at[pl.ds(base, SUB)], y_v[p], ld_sem[p])
        if not first:
            pltpu.async_copy(pp_hbm.at[0, pl.ds(base, SUB)], a_v[p],
                             ld_sem[p])
            pltpu.async_copy(pp_hbm.at[1, pl.ds(base, SUB)], b_v[p],
                             ld_sem[p])

    def wait_loads(p):
        pltpu.make_async_copy(dp_hbm.at[0, pl.ds(0, SUB)], dp0[p],
                              ld_sem[p]).wait()
        pltpu.make_async_copy(dp_hbm.at[1, pl.ds(0, SUB)], dp1[p],
                              ld_sem[p]).wait()
        pltpu.make_async_copy(y_hbm.at[pl.ds(0, SUB)], y_v[p],
                              ld_sem[p]).wait()
        if not first:
            pltpu.make_async_copy(pp_hbm.at[0, pl.ds(0, SUB)], a_v[p],
                                  ld_sem[p]).wait()
            pltpu.make_async_copy(pp_hbm.at[1, pl.ds(0, SUB)], b_v[p],
                                  ld_sem[p]).wait()

    def wait_store(p):
        pltpu.make_async_copy(a_v[p], gout_hbm.at[cid, pl.ds(0, SUB)],
                              st_sem[p]).wait()

    loads(0, 0)
    loads(1, 1)
    for s in range(nsub):
        p = s % 2
        wait_loads(p)
        if s >= 2:
            wait_store(p)

        def row(r4, carry):
            for k in range(UNR):
                r = r4 * UNR + k
                cnt = dp0[p][r, :] + dp1[p][r, :] + 1.0
                d = _fast_rsqrt(cnt)
                if first:
                    for c in range(DP // L):
                        sl = pl.ds(c * L, L)
                        a_v[p][r, sl] = d * y_v[p][r, sl]
                else:
                    t = d * d
                    u = t * d
                    for c in range(DP // L):
                        sl = pl.ds(c * L, L)
                        a_v[p][r, sl] = (t * (a_v[p][r, sl] + b_v[p][r, sl])
                                         + u * y_v[p][r, sl])
            return carry

        lax.fori_loop(0, SUB // UNR, row, 0)
        pltpu.async_copy(a_v[p], gout_hbm.at[cid, pl.ds(sub_base(s), SUB)],
                         st_sem[p])
        if s + 2 < nsub:
            loads(s + 2, p)
    wait_store(nsub % 2)
    wait_store((nsub + 1) % 2)

    for q in range(RPS // B):
        pltpu.make_async_copy(rows[0], acc.at[pl.ds(0, B)], zsem).wait()
    pltpu.make_async_copy(ei_hbm.at[0, pl.ds(0, NCH)], src_all,
                          misc_sem).wait()
    pltpu.make_async_copy(ei_hbm.at[1, pl.ds(0, NCH)], dst_all,
                          misc_sem).wait()

    plsc.subcore_barrier()

    _edge_pipeline(acc, src_all, dst_all, rows, gsem, ssem, gout_hbm.at[cid])

    @pl.when(w < ECH - NW * NCH)
    def _extra():
        pltpu.sync_copy(ei_hbm.at[0, NW * NCH + w], xsrc)
        pltpu.sync_copy(ei_hbm.at[1, NW * NCH + w], xdst)
        pltpu.async_copy(gout_hbm.at[cid].at[xsrc], rows[0], gsem[0])
        pltpu.make_async_copy(gout_hbm.at[cid].at[xsrc], rows[0],
                              gsem[0]).wait()
        pltpu.sync_copy(rows[0], acc.at[xdst], add=True)

    plsc.subcore_barrier()

    pltpu.sync_copy(acc.at[pl.ds(sid * RPS, RPS)],
                    parts_hbm.at[cid, pl.ds(sid * RPS, RPS)])


_hop_ring_scratch = (
    [pltpu.VMEM((B, DP), jnp.float32)] * NBUF
    + [pltpu.SemaphoreType.DMA] * (2 * NBUF))

def _pair(shape, dtype):
    return (pltpu.VMEM(shape, dtype), pltpu.VMEM(shape, dtype))


_SEM_PAIR = (pltpu.SemaphoreType.DMA, pltpu.SemaphoreType.DMA)

_hop1 = pl.kernel(
    _hop1_body,
    out_type=(jax.ShapeDtypeStruct((NC, NP, DP), jnp.float32),
              jax.ShapeDtypeStruct((NC, NP, DP), jnp.float32)),
    mesh=_MESH,
    scratch_types=[
        pltpu.VMEM_SHARED((NP, DP), jnp.float32),
        pltpu.VMEM((NCH, B), jnp.int32),
        pltpu.VMEM((NCH, B), jnp.int32),
        pltpu.VMEM((B,), jnp.int32),
        pltpu.VMEM((B,), jnp.int32),
        _pair((SUB, DP), jnp.float32),
        _pair((SUB, DP), jnp.float32),
        _pair((SUB, DEGW), jnp.float32),
        _pair((SUB, DEGW), jnp.float32),
        pltpu.SemaphoreType.DMA,
        pltpu.SemaphoreType.DMA,
        _SEM_PAIR,
        _SEM_PAIR,
    ] + _hop_ring_scratch,
    compiler_params=_SC_PARAMS,
)

_hop2 = pl.kernel(
    _hop2_body,
    out_type=(jax.ShapeDtypeStruct((NC, NP, DP), jnp.float32),
              jax.ShapeDtypeStruct((NC, NP, DP), jnp.float32)),
    mesh=_MESH,
    scratch_types=[
        pltpu.VMEM_SHARED((NP, DP), jnp.float32),
        pltpu.VMEM((NCH, B), jnp.int32),
        pltpu.VMEM((NCH, B), jnp.int32),
        pltpu.VMEM((B,), jnp.int32),
        pltpu.VMEM((B,), jnp.int32),
        _pair((SUB, DP), jnp.float32),
        _pair((SUB, DP), jnp.float32),
        _pair((SUB, DP), jnp.float32),
        _pair((SUB, DEGW), jnp.float32),
        _pair((SUB, DEGW), jnp.float32),
        pltpu.SemaphoreType.DMA,
        pltpu.SemaphoreType.DMA,
        _SEM_PAIR,
        _SEM_PAIR,
    ] + _hop_ring_scratch,
    compiler_params=_SC_PARAMS,
)


def _deg_body(ei_hbm, out_hbm, acc, dst_all, xdst, ones_v, sem, zsem):
    cid = lax.axis_index("c")
    sid = lax.axis_index("s")
    w = cid * NS + sid

    pltpu.async_copy(ei_hbm.at[1, pl.ds(w * NCH, NCH)], dst_all, sem)
    _zero_vmem(ones_v, B, DEGW // L)
    for q in range(RPS // B):
        pltpu.async_copy(ones_v, acc.at[pl.ds(sid * RPS + q * B, B)], zsem)
    for q in range(RPS // B):
        pltpu.make_async_copy(ones_v, acc.at[pl.ds(0, B)], zsem).wait()
    ov = jnp.full((L,), 1.0, jnp.float32)

    def orow(r, carry):
        ones_v[r, pl.ds(0, L)] = ov
        return carry

    lax.fori_loop(0, B, orow, 0)
    pltpu.make_async_copy(ei_hbm.at[1, pl.ds(0, NCH)], dst_all, sem).wait()
    plsc.subcore_barrier()

    def issue(j):
        pltpu.async_copy(ones_v, acc.at[dst_all.at[j]], sem, add=True)

    def drain_one():
        pltpu.make_async_copy(ones_v, acc.at[dst_all.at[0]], sem).wait()

    for j in range(DEG_LAG):
        issue(j)

    def step(j, carry):
        issue(j)
        drain_one()
        return carry

    lax.fori_loop(DEG_LAG, NCH, step, 0)
    for _ in range(DEG_LAG):
        drain_one()

    @pl.when(w < ECH - NW * NCH)
    def _extra():
        pltpu.sync_copy(ei_hbm.at[1, NW * NCH + w], xdst)
        pltpu.sync_copy(ones_v, acc.at[xdst], add=True)

    plsc.subcore_barrier()
    pltpu.sync_copy(acc.at[pl.ds(sid * RPS, RPS)],
                    out_hbm.at[cid, pl.ds(sid * RPS, RPS)])


_deg = pl.kernel(
    _deg_body,
    out_type=jax.ShapeDtypeStruct((NC, NP, DEGW), jnp.float32),
    mesh=_MESH,
    scratch_types=[
        pltpu.VMEM_SHARED((NP, DEGW), jnp.float32),
        pltpu.VMEM((NCH, B), jnp.int32),
        pltpu.VMEM((B,), jnp.int32),
        pltpu.VMEM((B, DEGW), jnp.float32),
        pltpu.SemaphoreType.DMA,
        pltpu.SemaphoreType.DMA,
    ],
    compiler_params=_SC_PARAMS,
)


def _finish_body(dp_hbm, pp_hbm, g_hbm, out_hbm,
                 dp0, dp1, p0_v, p1_v, g_v, o_v, sem):
    cid = lax.axis_index("c")
    sid = lax.axis_index("s")
    w = cid * NS + sid
    base = w * RPW

    pairs = [(dp_hbm.at[0, pl.ds(base, RPW)], dp0),
             (dp_hbm.at[1, pl.ds(base, RPW)], dp1),
             (pp_hbm.at[0, pl.ds(base, RPW)], p0_v),
             (pp_hbm.at[1, pl.ds(base, RPW)], p1_v),
             (g_hbm.at[0, pl.ds(base, RPW)], g_v)]
    for s, d in pairs:
        pltpu.async_copy(s, d, sem)
    for s, d in pairs:
        pltpu.make_async_copy(s, d, sem).wait()

    def row(r4, carry):
        for k in range(UNR):
            r = r4 * UNR + k
            cnt = dp0[r, :] + dp1[r, :] + 1.0
            d = _fast_rsqrt(cnt)
            for c in range(DP // L):
                sl = pl.ds(c * L, L)
                o_v[r, sl] = d * (p0_v[r, sl] + p1_v[r, sl] + g_v[r, sl])
        return carry

    lax.fori_loop(0, RPW // UNR, row, 0)
    pltpu.sync_copy(o_v, out_hbm.at[pl.ds(base, RPW)])


_finish = pl.kernel(
    _finish_body,
    out_type=jax.ShapeDtypeStruct((NP, DP), jnp.float32),
    mesh=_MESH,
    scratch_types=[
        pltpu.VMEM((RPW, DEGW), jnp.float32),
        pltpu.VMEM((RPW, DEGW), jnp.float32),
        pltpu.VMEM((RPW, DP), jnp.float32),
        pltpu.VMEM((RPW, DP), jnp.float32),
        pltpu.VMEM((RPW, DP), jnp.float32),
        pltpu.VMEM((RPW, DP), jnp.float32),
        pltpu.SemaphoreType.DMA,
    ],
    compiler_params=_SC_PARAMS,
)


def _mm_body(x_ref, w_ref, y_ref):
    y_ref[pl.ds(0, N), :] = jnp.dot(x_ref[...], w_ref[...],
                                    preferred_element_type=jnp.float32)


_mm = pl.pallas_call(
    _mm_body,
    out_shape=jax.ShapeDtypeStruct((NP, DP), jnp.float32),
)


@jax.jit
def kernel(x, edge_index, W):
    ei3 = edge_index.astype(jnp.int32).reshape(2, ECH, B)

    Wp = jnp.pad(W, ((0, 0), (0, DP - C)))

    deg_parts = _deg(ei3)
    y = _mm(x, Wp)
    parts1, _ = _hop1(deg_parts, y, ei3)
    parts2, g2r = _hop2(deg_parts, y, parts1, ei3)
    outp = _finish(deg_parts, parts2, g2r)
    return outp[:N, :C]
